# baseline (speedup 1.0000x reference)
import functools

import jax
import jax.numpy as jnp
from jax import lax
from jax.experimental import pallas as pl
from jax.experimental.pallas import tpu as pltpu
from jax.experimental.pallas import tpu_sc as plsc

_N = 10000
_E = 320000
_D = 128
_H = 64
_C = 40

_NW = 32
_CH = 128
_NCH = 80
_GRP = 8
_NGRP = _NCH // _GRP
_EPW = _NCH * _CH
_EPAD = _NW * _EPW
_NP = 10240
_HP = 128
_RPT = 640


def _make_sc_body(width, nb, seed_x):
    def body_fn(x_hbm, src_hbm, dst_hbm, out_hbm,
                sidx_v, didx_v, rows_v, agg_sp, gsem):
        c = lax.axis_index("c")
        s = lax.axis_index("s")
        w = s * 2 + c
        r0 = s * _RPT
        pltpu.sync_copy(x_hbm.at[pl.ds(r0, _RPT)],
                        agg_sp.at[pl.ds(r0, _RPT)])
        plsc.subcore_barrier()

        pltpu.sync_copy(src_hbm.at[w], sidx_v)
        pltpu.sync_copy(dst_hbm.at[w], didx_v)

        def body(ci, carry):
            pltpu.async_copy(
                x_hbm.at[sidx_v.at[ci]], rows_v, gsem).wait()
            pltpu.sync_copy(rows_v, agg_sp.at[didx_v.at[ci]], add=True)
            return carry

        lax.fori_loop(0, _NCH, body, 0)
        plsc.subcore_barrier()
        pltpu.sync_copy(agg_sp.at[pl.ds(r0, _RPT)],
                        out_hbm.at[c, pl.ds(r0, _RPT)])

    return body_fn


def _make_sc_segsum(width, nb, seed_x):
    return functools.partial(
        pl.kernel,
        _make_sc_body(width, nb, seed_x),
        out_type=jax.ShapeDtypeStruct((2, _NP, width), jnp.float32),
        mesh=plsc.VectorSubcoreMesh(core_axis_name="c", subcore_axis_name="s"),
        scratch_types=[
            pltpu.VMEM((_NCH, _CH), jnp.int32),
            pltpu.VMEM((_NCH, _CH), jnp.int32),
            pltpu.VMEM((_CH, _HP), jnp.float32),
            pltpu.VMEM_SHARED((_NP, width), jnp.float32),
            pltpu.SemaphoreType.DMA,
        ],
    )()


_sc_segsum_w = _make_sc_segsum(_HP, 2, True)


def _tc_gin1_body(x_ref, p_ref, wa_ref, ba_ref, wb_ref, bb_ref, o_ref):
    k = wa_ref.shape[0]
    t = p_ref[0, :, :k] + p_ref[1, :, :k] - x_ref[:, :k]
    h = jnp.dot(t, wa_ref[...], preferred_element_type=jnp.float32)
    h = jnp.maximum(h + ba_ref[...], 0.0)
    g = jnp.dot(h, wb_ref[...], preferred_element_type=jnp.float32)
    g = jnp.maximum(g + bb_ref[...], 0.0)
    o_ref[...] = jnp.concatenate([g, jnp.zeros_like(g)], axis=1)


def _tc_gin2_body(x_ref, p_ref, wa_ref, ba_ref, wb_ref, bb_ref, o_ref):
    k = wa_ref.shape[0]
    t = p_ref[0, :, :k] + p_ref[1, :, :k] - x_ref[:, :k]
    h = jnp.dot(t, wa_ref[...], preferred_element_type=jnp.float32)
    h = jnp.maximum(h + ba_ref[...], 0.0)
    g = jnp.dot(h, wb_ref[...], preferred_element_type=jnp.float32)
    g = jnp.maximum(g + bb_ref[...], 0.0)
    o_ref[...] = jnp.concatenate([g, jnp.zeros_like(g)], axis=1)


def _tc_fin_body(x_ref, p_ref, wa_ref, ba_ref, wb_ref, bb_ref, wl_ref, bl_ref,
                 o_ref):
    t = p_ref[0, :, :_H] + p_ref[1, :, :_H] - x_ref[:, :_H]
    h = jnp.dot(t, wa_ref[...], preferred_element_type=jnp.float32)
    h = jnp.maximum(h + ba_ref[...], 0.0)
    g = jnp.dot(h, wb_ref[...], preferred_element_type=jnp.float32)
    g = jnp.maximum(g + bb_ref[...], 0.0)
    o_ref[...] = jnp.dot(g, wl_ref[...],
                         preferred_element_type=jnp.float32) + bl_ref[...]


_tc_gin1 = pl.pallas_call(
    _tc_gin1_body, out_shape=jax.ShapeDtypeStruct((_NP, _HP), jnp.float32))
_tc_gin2 = pl.pallas_call(
    _tc_gin2_body, out_shape=jax.ShapeDtypeStruct((_NP, _HP), jnp.float32))
_tc_fin = pl.pallas_call(
    _tc_fin_body, out_shape=jax.ShapeDtypeStruct((_NP, _C), jnp.float32))


def kernel(features, edge_indicies, W1a, b1a, W1b, b1b, W2a, b2a, W2b, b2b,
           W3a, b3a, W3b, b3b, Wl, bl):
    src = edge_indicies[0]
    dst = edge_indicies[1]
    pad = _EPAD - _E
    src_p = jnp.concatenate(
        [src, jnp.zeros((pad,), jnp.int32)]).reshape(_NW, _NCH, _CH)
    dst_p = jnp.concatenate(
        [dst, jnp.full((pad,), _N, jnp.int32)]).reshape(_NW, _NCH, _CH)

    x0 = jnp.pad(features, ((0, _NP - _N), (0, 0)))
    p1 = _sc_segsum_w(x0, src_p, dst_p)
    x1 = _tc_gin1(x0, p1, W1a, b1a.reshape(1, _H), W1b, b1b.reshape(1, _H))
    p2 = _sc_segsum_w(x1, src_p, dst_p)
    x2 = _tc_gin2(x1, p2, W2a, b2a.reshape(1, _H), W2b, b2b.reshape(1, _H))
    p3 = _sc_segsum_w(x2, src_p, dst_p)
    out = _tc_fin(x2, p3, W3a, b3a.reshape(1, _H), W3b, b3b.reshape(1, _H),
                  Wl, bl.reshape(1, _C))
    return out[:_N]


# speedup vs baseline: 1.4707x; 1.4707x over previous
import functools

import jax
import jax.numpy as jnp
from jax import lax
from jax.experimental import pallas as pl
from jax.experimental.pallas import tpu as pltpu
from jax.experimental.pallas import tpu_sc as plsc

_N = 10000
_E = 320000
_D = 128
_H = 64
_C = 40

_NW = 32
_CH = 128
_NCH = 79
_GRP = 8
_NGRP = _NCH // _GRP
_EPW = _NCH * _CH
_EPAD = _NW * _EPW
_NP = 10240
_HP = 128
_RPT = 640


def _make_sc_body(width, nb, seed_x):
    def body_fn(x_hbm, src_hbm, dst_hbm, out_hbm,
                sidx_v, didx_v, rows_v, agg_sp, gsem):
        c = lax.axis_index("c")
        s = lax.axis_index("s")
        w = s * 2 + c
        r0 = s * _RPT
        pltpu.sync_copy(x_hbm.at[pl.ds(r0, _RPT)],
                        agg_sp.at[pl.ds(r0, _RPT)])
        pltpu.sync_copy(src_hbm.at[w], sidx_v)
        pltpu.sync_copy(dst_hbm.at[w], didx_v)
        plsc.subcore_barrier()

        def body(ci, carry):
            pltpu.async_copy(
                x_hbm.at[sidx_v.at[ci]], rows_v, gsem).wait()
            pltpu.sync_copy(rows_v, agg_sp.at[didx_v.at[ci]], add=True)
            return carry

        lax.fori_loop(0, _NCH, body, 0)
        plsc.subcore_barrier()
        pltpu.sync_copy(agg_sp.at[pl.ds(r0, _RPT)],
                        out_hbm.at[c, pl.ds(r0, _RPT)])

    return body_fn


def _make_sc_segsum(width, nb, seed_x):
    return functools.partial(
        pl.kernel,
        _make_sc_body(width, nb, seed_x),
        out_type=jax.ShapeDtypeStruct((2, _NP, width), jnp.float32),
        mesh=plsc.VectorSubcoreMesh(core_axis_name="c", subcore_axis_name="s"),
        scratch_types=[
            pltpu.VMEM((_NCH, _CH), jnp.int32),
            pltpu.VMEM((_NCH, _CH), jnp.int32),
            pltpu.VMEM((_CH, _HP), jnp.float32),
            pltpu.VMEM_SHARED((_NP, width), jnp.float32),
            pltpu.SemaphoreType.DMA,
        ],
    )()


_sc_segsum_w = _make_sc_segsum(_HP, 2, True)


def _tc_gin1_body(x_ref, p_ref, wa_ref, ba_ref, wb_ref, bb_ref, o_ref):
    k = wa_ref.shape[0]
    t = p_ref[0, :, :k] + p_ref[1, :, :k] - x_ref[:, :k]
    h = jnp.dot(t, wa_ref[...], preferred_element_type=jnp.float32)
    h = jnp.maximum(h + ba_ref[...], 0.0)
    g = jnp.dot(h, wb_ref[...], preferred_element_type=jnp.float32)
    g = jnp.maximum(g + bb_ref[...], 0.0)
    o_ref[...] = jnp.concatenate([g, jnp.zeros_like(g)], axis=1)


def _tc_gin2_body(x_ref, p_ref, wa_ref, ba_ref, wb_ref, bb_ref, o_ref):
    k = wa_ref.shape[0]
    t = p_ref[0, :, :k] + p_ref[1, :, :k] - x_ref[:, :k]
    h = jnp.dot(t, wa_ref[...], preferred_element_type=jnp.float32)
    h = jnp.maximum(h + ba_ref[...], 0.0)
    g = jnp.dot(h, wb_ref[...], preferred_element_type=jnp.float32)
    g = jnp.maximum(g + bb_ref[...], 0.0)
    o_ref[...] = jnp.concatenate([g, jnp.zeros_like(g)], axis=1)


def _tc_fin_body(x_ref, p_ref, wa_ref, ba_ref, wb_ref, bb_ref, wl_ref, bl_ref,
                 o_ref):
    t = p_ref[0, :, :_H] + p_ref[1, :, :_H] - x_ref[:, :_H]
    h = jnp.dot(t, wa_ref[...], preferred_element_type=jnp.float32)
    h = jnp.maximum(h + ba_ref[...], 0.0)
    g = jnp.dot(h, wb_ref[...], preferred_element_type=jnp.float32)
    g = jnp.maximum(g + bb_ref[...], 0.0)
    o_ref[...] = jnp.dot(g, wl_ref[...],
                         preferred_element_type=jnp.float32) + bl_ref[...]


_tc_gin1 = pl.pallas_call(
    _tc_gin1_body, out_shape=jax.ShapeDtypeStruct((_NP, _HP), jnp.float32))
_tc_gin2 = pl.pallas_call(
    _tc_gin2_body, out_shape=jax.ShapeDtypeStruct((_NP, _HP), jnp.float32))
_tc_fin = pl.pallas_call(
    _tc_fin_body, out_shape=jax.ShapeDtypeStruct((_NP, _C), jnp.float32))


def kernel(features, edge_indicies, W1a, b1a, W1b, b1b, W2a, b2a, W2b, b2b,
           W3a, b3a, W3b, b3b, Wl, bl):
    src = edge_indicies[0]
    dst = edge_indicies[1]
    pad = _EPAD - _E
    src_p = jnp.concatenate(
        [src, jnp.zeros((pad,), jnp.int32)]).reshape(_NW, _NCH, _CH)
    dst_p = jnp.concatenate(
        [dst, jnp.full((pad,), _N, jnp.int32)]).reshape(_NW, _NCH, _CH)

    x0 = jnp.pad(features, ((0, _NP - _N), (0, 0)))
    p1 = _sc_segsum_w(x0, src_p, dst_p)
    x1 = _tc_gin1(x0, p1, W1a, b1a.reshape(1, _H), W1b, b1b.reshape(1, _H))
    p2 = _sc_segsum_w(x1, src_p, dst_p)
    x2 = _tc_gin2(x1, p2, W2a, b2a.reshape(1, _H), W2b, b2b.reshape(1, _H))
    p3 = _sc_segsum_w(x2, src_p, dst_p)
    out = _tc_fin(x2, p3, W3a, b3a.reshape(1, _H), W3b, b3b.reshape(1, _H),
                  Wl, bl.reshape(1, _C))
    return out[:_N]
